# baseline (speedup 1.0000x reference)
import functools

import jax
import jax.numpy as jnp
import numpy as np
from jax import lax
from jax.experimental import pallas as pl
from jax.experimental.pallas import tpu as pltpu
from jax.experimental.pallas import tpu_sc as plsc

_C = 2
_N = 2048
_NT = 3
_IF = 128
_OF = 32
_H = 4
_CN = _C * _N
_F = _NT * _OF
_FP = 128
_K = _N * _H
_E = 65536

_NC = 2
_NS = 16
_NW = _NC * _NS
_EPW = _E // _NW

_V = 128


def _perm_matrices():
    p2 = np.zeros((_NT * _IF, _H * _FP), np.float32)
    for h in range(_H):
        for nt in range(_NT):
            for of in range(_OF):
                src = nt * 128 + h * 32 + of
                dst = h * _FP + nt * _OF + of
                p2[src, dst] = 1.0
    s2 = np.zeros((_H * _FP, _H), np.float32)
    for j in range(_H * _FP):
        s2[j, j // _FP] = 1.0
    return jnp.asarray(p2), jnp.asarray(s2)


def _pad_attn(attn):
    return jnp.pad(attn.reshape(1, _H, _F), ((0, 0), (0, 0), (0, _FP - _F))
                   ).reshape(1, _H * _FP)


def _sc_masks_body(edge_hbm, out_hbm, idx_s, idx_d, fs, fd):
    wid = lax.axis_index("s") * _NC + lax.axis_index("c")
    base = wid * _EPW
    pltpu.sync_copy(edge_hbm.at[0, pl.ds(base, _EPW)], idx_s)
    pltpu.sync_copy(edge_hbm.at[1, pl.ds(base, _EPW)], idx_d)

    zeros = jnp.zeros((16,), jnp.float32)

    def zbody(i, carry):
        fs[pl.ds(i * 16, 16)] = zeros
        fd[pl.ds(i * 16, 16)] = zeros
        return carry

    lax.fori_loop(0, _CN // 16, zbody, 0)

    ones = jnp.ones((16,), jnp.float32)

    def sbody(i, carry):
        vi = idx_s[pl.ds(i * 16, 16)]
        plsc.store_scatter(fs, [vi], ones)
        vj = idx_d[pl.ds(i * 16, 16)]
        plsc.store_scatter(fd, [vj], ones)
        return carry

    lax.fori_loop(0, _EPW // 16, sbody, 0)

    pltpu.sync_copy(fs, out_hbm.at[0, wid])
    pltpu.sync_copy(fd, out_hbm.at[1, wid])


def _sc_mask_partials(edge_idx):
    mesh = plsc.VectorSubcoreMesh(core_axis_name="c", subcore_axis_name="s")
    fn = functools.partial(
        pl.kernel,
        out_type=jax.ShapeDtypeStruct((2, _NW, _CN), jnp.float32),
        mesh=mesh,
        scratch_types=[
            pltpu.VMEM((_EPW,), jnp.int32),
            pltpu.VMEM((_EPW,), jnp.int32),
            pltpu.VMEM((_CN,), jnp.float32),
            pltpu.VMEM((_CN,), jnp.float32),
        ],
        compiler_params=pltpu.CompilerParams(needs_layout_passes=False),
    )(_sc_masks_body)
    return fn(edge_idx)


def _feat_body(h_ref, fc_ref, o_ref):
    for nt in range(_NT):
        o_ref[0, nt] = jnp.dot(
            h_ref[:, nt, :], fc_ref[nt], preferred_element_type=jnp.float32
        )


def _feat_kernel(h_src, fc):
    blk = 1024
    nb = _N // blk
    return pl.pallas_call(
        _feat_body,
        grid=(_CN // blk,),
        in_specs=[
            pl.BlockSpec((blk, _NT, _IF), lambda i: (i, 0, 0)),
            pl.BlockSpec((_NT, _IF, _OF * _H), lambda i: (0, 0, 0)),
        ],
        out_specs=pl.BlockSpec((1, _NT, blk, _OF * _H),
                               lambda i: (i // nb, 0, i % nb, 0)),
        out_shape=jax.ShapeDtypeStruct((_C, _NT, _N, _OF * _H), jnp.float32),
    )(h_src, fc)


def _prep_body(part_ref, f_ref, p2_ref, s2_ref, as_ref, ad_ref,
               asrc_ref, adst_ref, adt_ref, fd_ref):
    part = part_ref[...]
    cnt = part.sum(axis=1)
    sflag = (cnt[0] > 0.0).astype(jnp.float32)
    dflag = (cnt[1] > 0.0).astype(jnp.float32)
    fp = jnp.dot(f_ref[...], p2_ref[...],
                 preferred_element_type=jnp.float32)
    ps = jnp.clip(fp * as_ref[...], -1e9, 1e9)
    asrc_ref[...] = jnp.dot(ps, s2_ref[...],
                            preferred_element_type=jnp.float32) * sflag[:, None]
    pd = jnp.clip(fp * ad_ref[...], -1e9, 1e9)
    adst = jnp.dot(pd, s2_ref[...],
                   preferred_element_type=jnp.float32) * dflag[:, None]
    adst_ref[...] = adst
    adt_ref[...] = adst.T
    fd_ref[...] = fp * dflag[:, None]


def _prep_kernel(partials, flat, p2, s2, attn_src_p, attn_dst_p):
    blk = 1024
    kp = _H * _FP
    return pl.pallas_call(
        _prep_body,
        grid=(_CN // blk,),
        in_specs=[
            pl.BlockSpec((2, _NW, blk), lambda i: (0, 0, i)),
            pl.BlockSpec((blk, _NT * _IF), lambda i: (i, 0)),
            pl.BlockSpec((_NT * _IF, kp), lambda i: (0, 0)),
            pl.BlockSpec((kp, _H), lambda i: (0, 0)),
            pl.BlockSpec((1, kp), lambda i: (0, 0)),
            pl.BlockSpec((1, kp), lambda i: (0, 0)),
        ],
        out_specs=[
            pl.BlockSpec((blk, _H), lambda i: (i, 0)),
            pl.BlockSpec((blk, _H), lambda i: (i, 0)),
            pl.BlockSpec((_H, blk), lambda i: (0, i)),
            pl.BlockSpec((blk, kp), lambda i: (i, 0)),
        ],
        out_shape=[
            jax.ShapeDtypeStruct((_CN, _H), jnp.float32),
            jax.ShapeDtypeStruct((_CN, _H), jnp.float32),
            jax.ShapeDtypeStruct((_H, _CN), jnp.float32),
            jax.ShapeDtypeStruct((_CN, kp), jnp.float32),
        ],
    )(partials, flat, p2, s2, attn_src_p, attn_dst_p)


def _lrelu(z):
    return jnp.maximum(z, 0.2 * z)


_KR = _K // 128


def _main_body(asrc_ref, b_ref, adt_ref, fd_ref, e_ref, a_ref, o_ref):
    asrc = asrc_ref[...]
    em = e_ref[...]
    b = b_ref[...]
    ze0 = jnp.dot(asrc[0], em, preferred_element_type=jnp.float32)
    ze1 = jnp.dot(asrc[1], em, preferred_element_type=jnp.float32)
    z0 = _lrelu(ze0[:, None, :] + b[0][None, :, :])
    z1 = _lrelu(ze1[:, None, :] + b[1][None, :, :])
    e0 = jnp.exp(z0)
    e1 = jnp.exp(z1)
    inv = 1.0 / (e0 + e1)
    a_ref[0] = (e0 * inv).reshape(_V * _KR, 128)
    a_ref[1] = (e1 * inv).reshape(_V * _KR, 128)

    acc0 = jnp.zeros((_V, _FP), jnp.float32)
    acc1 = jnp.zeros((_V, _FP), jnp.float32)
    for h in range(_H):
        zh0 = _lrelu(asrc[0, :, h][:, None] + adt_ref[h, 0:_N][None, :])
        zh1 = _lrelu(asrc[1, :, h][:, None] + adt_ref[h, _N:2 * _N][None, :])
        eh0 = jnp.exp(zh0)
        eh1 = jnp.exp(zh1)
        invh = 1.0 / (eh0 + eh1)
        fd0 = fd_ref[0, :, _FP * h:_FP * (h + 1)]
        fd1 = fd_ref[1, :, _FP * h:_FP * (h + 1)]
        acc0 = acc0 + jax.nn.relu(
            jnp.dot(eh0 * invh, fd0, preferred_element_type=jnp.float32))
        acc1 = acc1 + jax.nn.relu(
            jnp.dot(eh1 * invh, fd1, preferred_element_type=jnp.float32))
    o_ref[0] = acc0[:, :_F]
    o_ref[1] = acc1[:, :_F]


def _main_kernel(asrc3, b3, adt, featd3, em):
    return pl.pallas_call(
        _main_body,
        grid=(_N // _V,),
        in_specs=[
            pl.BlockSpec((_C, _V, _H), lambda i: (0, i, 0)),
            pl.BlockSpec((_C, _KR, 128), lambda i: (0, 0, 0)),
            pl.BlockSpec((_H, _CN), lambda i: (0, 0)),
            pl.BlockSpec((_C, _N, _H * _FP), lambda i: (0, 0, 0)),
            pl.BlockSpec((_H, 128), lambda i: (0, 0)),
        ],
        out_specs=[
            pl.BlockSpec((_C, _V * _KR, 128), lambda i: (0, i, 0)),
            pl.BlockSpec((_C, _V, _F), lambda i: (0, i, 0)),
        ],
        out_shape=[
            jax.ShapeDtypeStruct((_C, _N * _KR, 128), jnp.float32),
            jax.ShapeDtypeStruct((_C, _N, _F), jnp.float32),
        ],
    )(asrc3, b3, adt, featd3, em)


def kernel(edge_idx, node_feats, fc, attn_src, attn_dst):
    edge_idx = edge_idx.astype(jnp.int32)

    partials = _sc_mask_partials(edge_idx)

    h_src = node_feats.reshape(_CN, _NT, _IF)
    feat4 = _feat_kernel(h_src, fc)
    flat = feat4.reshape(_CN, _NT * _IF)

    p2, s2 = _perm_matrices()
    asrc, adst, adt, featd3 = _prep_kernel(
        partials, flat, p2, s2, _pad_attn(attn_src), _pad_attn(attn_dst))

    asrc3 = asrc.reshape(_C, _N, _H)
    b3 = adst.reshape(_C, _KR, 128)
    featd3 = featd3.reshape(_C, _N, _H * _FP)

    em = (jax.lax.broadcasted_iota(jnp.int32, (_H, 128), 1) % _H
          == jax.lax.broadcasted_iota(jnp.int32, (_H, 128), 0)
          ).astype(jnp.float32)

    a3, outs = _main_kernel(asrc3, b3, adt, featd3, em)

    a = a3.reshape(_C, _N, _N, _H)
    outs4 = outs.reshape(_C, _N, _NT, _OF)
    return (outs4, a)


# speedup vs baseline: 13.0902x; 13.0902x over previous
import functools

import jax
import jax.numpy as jnp
import numpy as np
from jax import lax
from jax.experimental import pallas as pl
from jax.experimental.pallas import tpu as pltpu
from jax.experimental.pallas import tpu_sc as plsc

_C = 2
_N = 2048
_NT = 3
_IF = 128
_OF = 32
_H = 4
_CN = _C * _N
_F = _NT * _OF
_FP = 128
_K = _N * _H
_E = 65536

_NC = 2
_NS = 16
_NW = _NC * _NS
_EPW = _E // _NW

_V = 128


def _perm_matrices():
    p2 = np.zeros((_NT * _IF, _H * _FP), np.float32)
    for h in range(_H):
        for nt in range(_NT):
            for of in range(_OF):
                src = nt * 128 + h * 32 + of
                dst = h * _FP + nt * _OF + of
                p2[src, dst] = 1.0
    s2 = np.zeros((_H * _FP, _H), np.float32)
    for j in range(_H * _FP):
        s2[j, j // _FP] = 1.0
    return jnp.asarray(p2), jnp.asarray(s2)


def _pad_attn(attn):
    return jnp.pad(attn.reshape(1, _H, _F), ((0, 0), (0, 0), (0, _FP - _F))
                   ).reshape(1, _H * _FP)


def _sc_masks_body(edge_hbm, out_hbm, idx_s, idx_d, fs, fd):
    wid = lax.axis_index("s") * _NC + lax.axis_index("c")
    base = wid * _EPW
    pltpu.sync_copy(edge_hbm.at[0, pl.ds(base, _EPW)], idx_s)
    pltpu.sync_copy(edge_hbm.at[1, pl.ds(base, _EPW)], idx_d)

    zeros = jnp.zeros((16,), jnp.float32)

    def zbody(i, carry):
        fs[pl.ds(i * 16, 16)] = zeros
        fd[pl.ds(i * 16, 16)] = zeros
        return carry

    lax.fori_loop(0, _CN // 16, zbody, 0)

    ones = jnp.ones((16,), jnp.float32)

    def sbody(i, carry):
        vi = idx_s[pl.ds(i * 16, 16)]
        plsc.store_scatter(fs, [vi], ones)
        vj = idx_d[pl.ds(i * 16, 16)]
        plsc.store_scatter(fd, [vj], ones)
        return carry

    lax.fori_loop(0, _EPW // 16, sbody, 0)

    pltpu.sync_copy(fs, out_hbm.at[0, wid])
    pltpu.sync_copy(fd, out_hbm.at[1, wid])


def _sc_mask_partials(edge_idx):
    mesh = plsc.VectorSubcoreMesh(core_axis_name="c", subcore_axis_name="s")
    fn = functools.partial(
        pl.kernel,
        out_type=jax.ShapeDtypeStruct((2, _NW, _CN), jnp.float32),
        mesh=mesh,
        scratch_types=[
            pltpu.VMEM((_EPW,), jnp.int32),
            pltpu.VMEM((_EPW,), jnp.int32),
            pltpu.VMEM((_CN,), jnp.float32),
            pltpu.VMEM((_CN,), jnp.float32),
        ],
        compiler_params=pltpu.CompilerParams(needs_layout_passes=False),
    )(_sc_masks_body)
    return fn(edge_idx)


def _feat_body(h_ref, fc_ref, o_ref):
    for nt in range(_NT):
        o_ref[0, nt] = jnp.dot(
            h_ref[:, nt, :], fc_ref[nt], preferred_element_type=jnp.float32
        )


def _feat_kernel(h_src, fc):
    blk = 1024
    nb = _N // blk
    return pl.pallas_call(
        _feat_body,
        grid=(_CN // blk,),
        in_specs=[
            pl.BlockSpec((blk, _NT, _IF), lambda i: (i, 0, 0)),
            pl.BlockSpec((_NT, _IF, _OF * _H), lambda i: (0, 0, 0)),
        ],
        out_specs=pl.BlockSpec((1, _NT, blk, _OF * _H),
                               lambda i: (i // nb, 0, i % nb, 0)),
        out_shape=jax.ShapeDtypeStruct((_C, _NT, _N, _OF * _H), jnp.float32),
    )(h_src, fc)


def _prep_body(part_ref, f_ref, p2_ref, s2_ref, as_ref, ad_ref,
               asrc_ref, adst_ref, adt_ref, fd_ref):
    part = part_ref[...]
    cnt = part.sum(axis=1)
    sflag = (cnt[0] > 0.0).astype(jnp.float32)
    dflag = (cnt[1] > 0.0).astype(jnp.float32)
    fp = jnp.dot(f_ref[...], p2_ref[...],
                 preferred_element_type=jnp.float32)
    ps = jnp.clip(fp * as_ref[...], -1e9, 1e9)
    asrc_ref[...] = jnp.dot(ps, s2_ref[...],
                            preferred_element_type=jnp.float32) * sflag[:, None]
    pd = jnp.clip(fp * ad_ref[...], -1e9, 1e9)
    adst = jnp.dot(pd, s2_ref[...],
                   preferred_element_type=jnp.float32) * dflag[:, None]
    adst_ref[...] = adst
    adt_ref[...] = adst.T
    fd_ref[...] = fp * dflag[:, None]


def _prep_kernel(partials, flat, p2, s2, attn_src_p, attn_dst_p):
    blk = 1024
    kp = _H * _FP
    return pl.pallas_call(
        _prep_body,
        grid=(_CN // blk,),
        in_specs=[
            pl.BlockSpec((2, _NW, blk), lambda i: (0, 0, i)),
            pl.BlockSpec((blk, _NT * _IF), lambda i: (i, 0)),
            pl.BlockSpec((_NT * _IF, kp), lambda i: (0, 0)),
            pl.BlockSpec((kp, _H), lambda i: (0, 0)),
            pl.BlockSpec((1, kp), lambda i: (0, 0)),
            pl.BlockSpec((1, kp), lambda i: (0, 0)),
        ],
        out_specs=[
            pl.BlockSpec((blk, _H), lambda i: (i, 0)),
            pl.BlockSpec((blk, _H), lambda i: (i, 0)),
            pl.BlockSpec((_H, blk), lambda i: (0, i)),
            pl.BlockSpec((blk, kp), lambda i: (i, 0)),
        ],
        out_shape=[
            jax.ShapeDtypeStruct((_CN, _H), jnp.float32),
            jax.ShapeDtypeStruct((_CN, _H), jnp.float32),
            jax.ShapeDtypeStruct((_H, _CN), jnp.float32),
            jax.ShapeDtypeStruct((_CN, kp), jnp.float32),
        ],
    )(partials, flat, p2, s2, attn_src_p, attn_dst_p)


def _lrelu(z):
    return jnp.maximum(z, 0.2 * z)


def _main_body(asrc_ref, b_ref, adt_ref, fd_ref, e_ref, a_ref, o_ref):
    asrc = asrc_ref[...]
    em = e_ref[...]
    b = b_ref[...]
    z0 = _lrelu(jnp.dot(asrc[0], em, preferred_element_type=jnp.float32)
                + b[0][None, :])
    z1 = _lrelu(jnp.dot(asrc[1], em, preferred_element_type=jnp.float32)
                + b[1][None, :])
    e0 = jnp.exp(z0)
    e1 = jnp.exp(z1)
    inv = 1.0 / (e0 + e1)
    a_ref[0] = e0 * inv
    a_ref[1] = e1 * inv

    acc0 = jnp.zeros((_V, _FP), jnp.float32)
    acc1 = jnp.zeros((_V, _FP), jnp.float32)
    for h in range(_H):
        zh0 = _lrelu(asrc[0, :, h][:, None] + adt_ref[h, 0:_N][None, :])
        zh1 = _lrelu(asrc[1, :, h][:, None] + adt_ref[h, _N:2 * _N][None, :])
        eh0 = jnp.exp(zh0)
        eh1 = jnp.exp(zh1)
        invh = 1.0 / (eh0 + eh1)
        fd0 = fd_ref[0, :, _FP * h:_FP * (h + 1)]
        fd1 = fd_ref[1, :, _FP * h:_FP * (h + 1)]
        acc0 = acc0 + jax.nn.relu(
            jnp.dot(eh0 * invh, fd0, preferred_element_type=jnp.float32))
        acc1 = acc1 + jax.nn.relu(
            jnp.dot(eh1 * invh, fd1, preferred_element_type=jnp.float32))
    o_ref[0] = acc0[:, :_F]
    o_ref[1] = acc1[:, :_F]


def _main_kernel(asrc3, b2, adt, featd3, em):
    return pl.pallas_call(
        _main_body,
        grid=(_N // _V,),
        in_specs=[
            pl.BlockSpec((_C, _V, _H), lambda i: (0, i, 0)),
            pl.BlockSpec((_C, _K), lambda i: (0, 0)),
            pl.BlockSpec((_H, _CN), lambda i: (0, 0)),
            pl.BlockSpec((_C, _N, _H * _FP), lambda i: (0, 0, 0)),
            pl.BlockSpec((_H, _K), lambda i: (0, 0)),
        ],
        out_specs=[
            pl.BlockSpec((_C, _V, _K), lambda i: (0, i, 0)),
            pl.BlockSpec((_C, _V, _F), lambda i: (0, i, 0)),
        ],
        out_shape=[
            jax.ShapeDtypeStruct((_C, _N, _K), jnp.float32),
            jax.ShapeDtypeStruct((_C, _N, _F), jnp.float32),
        ],
    )(asrc3, b2, adt, featd3, em)


def kernel(edge_idx, node_feats, fc, attn_src, attn_dst):
    edge_idx = edge_idx.astype(jnp.int32)

    partials = _sc_mask_partials(edge_idx)

    h_src = node_feats.reshape(_CN, _NT, _IF)
    feat4 = _feat_kernel(h_src, fc)
    flat = feat4.reshape(_CN, _NT * _IF)

    p2, s2 = _perm_matrices()
    asrc, adst, adt, featd3 = _prep_kernel(
        partials, flat, p2, s2, _pad_attn(attn_src), _pad_attn(attn_dst))

    asrc3 = asrc.reshape(_C, _N, _H)
    b2 = adst.reshape(_C, _K)
    featd3 = featd3.reshape(_C, _N, _H * _FP)

    em = (jax.lax.broadcasted_iota(jnp.int32, (_H, _K), 1) % _H
          == jax.lax.broadcasted_iota(jnp.int32, (_H, _K), 0)
          ).astype(jnp.float32)

    a3, outs = _main_kernel(asrc3, b2, adt, featd3, em)

    a = a3.reshape(_C, _N, _N, _H)
    outs4 = outs.reshape(_C, _N, _NT, _OF)
    return (outs4, a)


# speedup vs baseline: 38.7693x; 2.9617x over previous
import functools

import jax
import jax.numpy as jnp
import numpy as np
from jax import lax
from jax.experimental import pallas as pl
from jax.experimental.pallas import tpu as pltpu
from jax.experimental.pallas import tpu_sc as plsc

_C = 2
_N = 2048
_NT = 3
_IF = 128
_OF = 32
_H = 4
_CN = _C * _N
_F = _NT * _OF
_FP = 128
_K = _N * _H
_E = 65536

_NC = 2
_NS = 16
_NW = _NC * _NS
_EPW = _E // _NW

_V = 128


def _perm_matrices():
    p2 = np.zeros((_NT * _IF, _H * _FP), np.float32)
    for h in range(_H):
        for nt in range(_NT):
            for of in range(_OF):
                src = nt * 128 + h * 32 + of
                dst = h * _FP + nt * _OF + of
                p2[src, dst] = 1.0
    s2 = np.zeros((_H * _FP, _H), np.float32)
    for j in range(_H * _FP):
        s2[j, j // _FP] = 1.0
    return jnp.asarray(p2), jnp.asarray(s2)


def _pad_attn(attn):
    return jnp.pad(attn.reshape(1, _H, _F), ((0, 0), (0, 0), (0, _FP - _F))
                   ).reshape(1, _H * _FP)


def _sc_masks_body(edge_hbm, out_hbm, idx_s, idx_d, fs, fd):
    wid = lax.axis_index("s") * _NC + lax.axis_index("c")
    base = wid * _EPW
    pltpu.sync_copy(edge_hbm.at[0, pl.ds(base, _EPW)], idx_s)
    pltpu.sync_copy(edge_hbm.at[1, pl.ds(base, _EPW)], idx_d)

    zeros = jnp.zeros((16,), jnp.float32)

    def zbody(i, carry):
        fs[pl.ds(i * 16, 16)] = zeros
        fd[pl.ds(i * 16, 16)] = zeros
        return carry

    lax.fori_loop(0, _CN // 16, zbody, 0)

    ones = jnp.ones((16,), jnp.float32)

    def sbody(i, carry):
        vi = idx_s[pl.ds(i * 16, 16)]
        plsc.store_scatter(fs, [vi], ones)
        vj = idx_d[pl.ds(i * 16, 16)]
        plsc.store_scatter(fd, [vj], ones)
        return carry

    lax.fori_loop(0, _EPW // 16, sbody, 0)

    pltpu.sync_copy(fs, out_hbm.at[0, wid])
    pltpu.sync_copy(fd, out_hbm.at[1, wid])


def _sc_mask_partials(edge_idx):
    mesh = plsc.VectorSubcoreMesh(core_axis_name="c", subcore_axis_name="s")
    fn = functools.partial(
        pl.kernel,
        out_type=jax.ShapeDtypeStruct((2, _NW, _CN), jnp.float32),
        mesh=mesh,
        scratch_types=[
            pltpu.VMEM((_EPW,), jnp.int32),
            pltpu.VMEM((_EPW,), jnp.int32),
            pltpu.VMEM((_CN,), jnp.float32),
            pltpu.VMEM((_CN,), jnp.float32),
        ],
        compiler_params=pltpu.CompilerParams(needs_layout_passes=False),
    )(_sc_masks_body)
    return fn(edge_idx)


def _feat_body(h_ref, fc_ref, o_ref):
    for nt in range(_NT):
        o_ref[0, nt] = jnp.dot(
            h_ref[:, nt, :], fc_ref[nt], preferred_element_type=jnp.float32
        )


def _feat_kernel(h_src, fc):
    blk = 1024
    nb = _N // blk
    return pl.pallas_call(
        _feat_body,
        grid=(_CN // blk,),
        in_specs=[
            pl.BlockSpec((blk, _NT, _IF), lambda i: (i, 0, 0)),
            pl.BlockSpec((_NT, _IF, _OF * _H), lambda i: (0, 0, 0)),
        ],
        out_specs=pl.BlockSpec((1, _NT, blk, _OF * _H),
                               lambda i: (i // nb, 0, i % nb, 0)),
        out_shape=jax.ShapeDtypeStruct((_C, _NT, _N, _OF * _H), jnp.float32),
    )(h_src, fc)


def _prep_body(part_ref, f_ref, p2_ref, s2_ref, as_ref, ad_ref,
               asrc_ref, adst_ref, adt_ref, fd_ref):
    part = part_ref[...]
    cnt = part.sum(axis=1)
    sflag = (cnt[0] > 0.0).astype(jnp.float32)
    dflag = (cnt[1] > 0.0).astype(jnp.float32)
    fp = jnp.dot(f_ref[...], p2_ref[...],
                 preferred_element_type=jnp.float32)
    ps = jnp.clip(fp * as_ref[...], -1e9, 1e9)
    asrc_ref[...] = jnp.dot(ps, s2_ref[...],
                            preferred_element_type=jnp.float32) * sflag[:, None]
    pd = jnp.clip(fp * ad_ref[...], -1e9, 1e9)
    adst = jnp.dot(pd, s2_ref[...],
                   preferred_element_type=jnp.float32) * dflag[:, None]
    adst_ref[...] = adst
    adt_ref[...] = adst.T
    fd_ref[...] = fp * dflag[:, None]


def _prep_kernel(partials, flat, p2, s2, attn_src_p, attn_dst_p):
    blk = 1024
    kp = _H * _FP
    return pl.pallas_call(
        _prep_body,
        grid=(_CN // blk,),
        in_specs=[
            pl.BlockSpec((2, _NW, blk), lambda i: (0, 0, i)),
            pl.BlockSpec((blk, _NT * _IF), lambda i: (i, 0)),
            pl.BlockSpec((_NT * _IF, kp), lambda i: (0, 0)),
            pl.BlockSpec((kp, _H), lambda i: (0, 0)),
            pl.BlockSpec((1, kp), lambda i: (0, 0)),
            pl.BlockSpec((1, kp), lambda i: (0, 0)),
        ],
        out_specs=[
            pl.BlockSpec((blk, _H), lambda i: (i, 0)),
            pl.BlockSpec((blk, _H), lambda i: (i, 0)),
            pl.BlockSpec((_H, blk), lambda i: (0, i)),
            pl.BlockSpec((blk, kp), lambda i: (i, 0)),
        ],
        out_shape=[
            jax.ShapeDtypeStruct((_CN, _H), jnp.float32),
            jax.ShapeDtypeStruct((_CN, _H), jnp.float32),
            jax.ShapeDtypeStruct((_H, _CN), jnp.float32),
            jax.ShapeDtypeStruct((_CN, kp), jnp.float32),
        ],
    )(partials, flat, p2, s2, attn_src_p, attn_dst_p)


def _lrelu(z):
    return jnp.maximum(z, 0.2 * z)


_UT = _N // 128
_R = _UT * _H


def _main_body(asrc_ref, bb_ref, adt_ref, fd_ref, e_ref, a_ref, o_ref):
    asrc = asrc_ref[...]
    em = e_ref[...]
    bb = bb_ref[...]
    ae0 = jnp.dot(asrc[0], em, preferred_element_type=jnp.float32)
    ae1 = jnp.dot(asrc[1], em, preferred_element_type=jnp.float32)
    z0 = _lrelu(ae0[:, :, None] + bb[0][None, :, :])
    z1 = _lrelu(ae1[:, :, None] + bb[1][None, :, :])
    e0 = jnp.exp(z0)
    e1 = jnp.exp(z1)
    inv = 1.0 / (e0 + e1)
    a_ref[0] = e0 * inv
    a_ref[1] = e1 * inv

    acc0 = jnp.zeros((_V, _FP), jnp.float32)
    acc1 = jnp.zeros((_V, _FP), jnp.float32)
    for h in range(_H):
        zh0 = _lrelu(asrc[0, :, h][:, None] + adt_ref[h, 0:_N][None, :])
        zh1 = _lrelu(asrc[1, :, h][:, None] + adt_ref[h, _N:2 * _N][None, :])
        eh0 = jnp.exp(zh0)
        eh1 = jnp.exp(zh1)
        invh = 1.0 / (eh0 + eh1)
        fd0 = fd_ref[0, :, _FP * h:_FP * (h + 1)]
        fd1 = fd_ref[1, :, _FP * h:_FP * (h + 1)]
        acc0 = acc0 + jax.nn.relu(
            jnp.dot(eh0 * invh, fd0, preferred_element_type=jnp.float32))
        acc1 = acc1 + jax.nn.relu(
            jnp.dot(eh1 * invh, fd1, preferred_element_type=jnp.float32))
    o_ref[0] = acc0[:, :_F]
    o_ref[1] = acc1[:, :_F]


def _main_kernel(asrc3, bb, adt, featd3, em):
    return pl.pallas_call(
        _main_body,
        grid=(_N // _V,),
        in_specs=[
            pl.BlockSpec((_C, _V, _H), lambda i: (0, i, 0)),
            pl.BlockSpec((_C, _R, 128), lambda i: (0, 0, 0)),
            pl.BlockSpec((_H, _CN), lambda i: (0, 0)),
            pl.BlockSpec((_C, _N, _H * _FP), lambda i: (0, 0, 0)),
            pl.BlockSpec((_H, _R), lambda i: (0, 0)),
        ],
        out_specs=[
            pl.BlockSpec((_C, _V, _R, 128), lambda i: (0, i, 0, 0)),
            pl.BlockSpec((_C, _V, _F), lambda i: (0, i, 0)),
        ],
        out_shape=[
            jax.ShapeDtypeStruct((_C, _N, _R, 128), jnp.float32),
            jax.ShapeDtypeStruct((_C, _N, _F), jnp.float32),
        ],
    )(asrc3, bb, adt, featd3, em)


def kernel(edge_idx, node_feats, fc, attn_src, attn_dst):
    edge_idx = edge_idx.astype(jnp.int32)

    partials = _sc_mask_partials(edge_idx)

    h_src = node_feats.reshape(_CN, _NT, _IF)
    feat4 = _feat_kernel(h_src, fc)
    flat = feat4.reshape(_CN, _NT * _IF)

    p2, s2 = _perm_matrices()
    asrc, adst, adt, featd3 = _prep_kernel(
        partials, flat, p2, s2, _pad_attn(attn_src), _pad_attn(attn_dst))

    asrc3 = asrc.reshape(_C, _N, _H)
    bb = adst.reshape(_C, _UT, 128, _H).transpose(0, 1, 3, 2).reshape(
        _C, _R, 128)
    featd3 = featd3.reshape(_C, _N, _H * _FP)

    em = (jax.lax.broadcasted_iota(jnp.int32, (_H, _R), 1) % _H
          == jax.lax.broadcasted_iota(jnp.int32, (_H, _R), 0)
          ).astype(jnp.float32)

    a4, outs = _main_kernel(asrc3, bb, adt, featd3, em)

    a = a4.reshape(_C, _N, _UT, _H, 128).transpose(0, 1, 2, 4, 3).reshape(
        _C, _N, _N, _H)
    outs4 = outs.reshape(_C, _N, _NT, _OF)
    return (outs4, a)


# speedup vs baseline: 39.1883x; 1.0108x over previous
import functools

import jax
import jax.numpy as jnp
import numpy as np
from jax import lax
from jax.experimental import pallas as pl
from jax.experimental.pallas import tpu as pltpu
from jax.experimental.pallas import tpu_sc as plsc

_C = 2
_N = 2048
_NT = 3
_IF = 128
_OF = 32
_H = 4
_CN = _C * _N
_F = _NT * _OF
_FP = 128
_K = _N * _H
_E = 65536

_NC = 2
_NS = 16
_NW = _NC * _NS
_EPW = _E // _NW

_V = 128


def _perm_matrices():
    p2 = np.zeros((_NT * _IF, _H * _FP), np.float32)
    for h in range(_H):
        for nt in range(_NT):
            for of in range(_OF):
                src = nt * 128 + h * 32 + of
                dst = h * _FP + nt * _OF + of
                p2[src, dst] = 1.0
    s2 = np.zeros((_H * _FP, _H), np.float32)
    for j in range(_H * _FP):
        s2[j, j // _FP] = 1.0
    return jnp.asarray(p2), jnp.asarray(s2)


def _pad_attn(attn):
    return jnp.pad(attn.reshape(1, _H, _F), ((0, 0), (0, 0), (0, _FP - _F))
                   ).reshape(1, _H * _FP)


def _sc_masks_body(edge_hbm, out_hbm, idx_s, idx_d, fs, fd):
    wid = lax.axis_index("s") * _NC + lax.axis_index("c")
    base = wid * _EPW
    pltpu.sync_copy(edge_hbm.at[0, pl.ds(base, _EPW)], idx_s)
    pltpu.sync_copy(edge_hbm.at[1, pl.ds(base, _EPW)], idx_d)

    zeros = jnp.zeros((16,), jnp.float32)

    def zbody(i, carry):
        fs[pl.ds(i * 16, 16)] = zeros
        fd[pl.ds(i * 16, 16)] = zeros
        return carry

    lax.fori_loop(0, _CN // 16, zbody, 0)

    ones = jnp.ones((16,), jnp.float32)

    def sbody(i, carry):
        vi = idx_s[pl.ds(i * 16, 16)]
        plsc.store_scatter(fs, [vi], ones)
        vj = idx_d[pl.ds(i * 16, 16)]
        plsc.store_scatter(fd, [vj], ones)
        return carry

    lax.fori_loop(0, _EPW // 16, sbody, 0)

    pltpu.sync_copy(fs, out_hbm.at[0, wid])
    pltpu.sync_copy(fd, out_hbm.at[1, wid])


def _sc_mask_partials(edge_idx):
    mesh = plsc.VectorSubcoreMesh(core_axis_name="c", subcore_axis_name="s")
    fn = functools.partial(
        pl.kernel,
        out_type=jax.ShapeDtypeStruct((2, _NW, _CN), jnp.float32),
        mesh=mesh,
        scratch_types=[
            pltpu.VMEM((_EPW,), jnp.int32),
            pltpu.VMEM((_EPW,), jnp.int32),
            pltpu.VMEM((_CN,), jnp.float32),
            pltpu.VMEM((_CN,), jnp.float32),
        ],
        compiler_params=pltpu.CompilerParams(needs_layout_passes=False),
    )(_sc_masks_body)
    return fn(edge_idx)


def _feat_body(h_ref, fc_ref, o_ref):
    for nt in range(_NT):
        o_ref[0, nt] = jnp.dot(
            h_ref[:, nt, :], fc_ref[nt], preferred_element_type=jnp.float32
        )


def _feat_kernel(h_src, fc):
    blk = 1024
    nb = _N // blk
    return pl.pallas_call(
        _feat_body,
        grid=(_CN // blk,),
        in_specs=[
            pl.BlockSpec((blk, _NT, _IF), lambda i: (i, 0, 0)),
            pl.BlockSpec((_NT, _IF, _OF * _H), lambda i: (0, 0, 0)),
        ],
        out_specs=pl.BlockSpec((1, _NT, blk, _OF * _H),
                               lambda i: (i // nb, 0, i % nb, 0)),
        out_shape=jax.ShapeDtypeStruct((_C, _NT, _N, _OF * _H), jnp.float32),
    )(h_src, fc)


def _prep_body(part_ref, f_ref, p2_ref, s2_ref, as_ref, ad_ref,
               asrc_ref, adst_ref, adt_ref, fd_ref):
    part = part_ref[...]
    cnt = part.sum(axis=1)
    sflag = (cnt[0] > 0.0).astype(jnp.float32)
    dflag = (cnt[1] > 0.0).astype(jnp.float32)
    fp = jnp.dot(f_ref[...], p2_ref[...],
                 preferred_element_type=jnp.float32)
    ps = jnp.clip(fp * as_ref[...], -1e9, 1e9)
    asrc_ref[...] = jnp.dot(ps, s2_ref[...],
                            preferred_element_type=jnp.float32) * sflag[:, None]
    pd = jnp.clip(fp * ad_ref[...], -1e9, 1e9)
    adst = jnp.dot(pd, s2_ref[...],
                   preferred_element_type=jnp.float32) * dflag[:, None]
    adst_ref[...] = adst
    adt_ref[...] = adst.T
    fd_ref[...] = fp * dflag[:, None]


def _prep_kernel(partials, flat, p2, s2, attn_src_p, attn_dst_p):
    blk = 1024
    kp = _H * _FP
    return pl.pallas_call(
        _prep_body,
        grid=(_CN // blk,),
        in_specs=[
            pl.BlockSpec((2, _NW, blk), lambda i: (0, 0, i)),
            pl.BlockSpec((blk, _NT * _IF), lambda i: (i, 0)),
            pl.BlockSpec((_NT * _IF, kp), lambda i: (0, 0)),
            pl.BlockSpec((kp, _H), lambda i: (0, 0)),
            pl.BlockSpec((1, kp), lambda i: (0, 0)),
            pl.BlockSpec((1, kp), lambda i: (0, 0)),
        ],
        out_specs=[
            pl.BlockSpec((blk, _H), lambda i: (i, 0)),
            pl.BlockSpec((blk, _H), lambda i: (i, 0)),
            pl.BlockSpec((_H, blk), lambda i: (0, i)),
            pl.BlockSpec((blk, kp), lambda i: (i, 0)),
        ],
        out_shape=[
            jax.ShapeDtypeStruct((_CN, _H), jnp.float32),
            jax.ShapeDtypeStruct((_CN, _H), jnp.float32),
            jax.ShapeDtypeStruct((_H, _CN), jnp.float32),
            jax.ShapeDtypeStruct((_CN, kp), jnp.float32),
        ],
    )(partials, flat, p2, s2, attn_src_p, attn_dst_p)


def _lrelu(z):
    return jnp.maximum(z, 0.2 * z)


_UT = _N // 128
_R = _UT * _H


def _main_body(asrc_ref, bb_ref, adt_ref, fd_ref, e_ref, a_ref, o_ref):
    asrc = asrc_ref[...]
    em = e_ref[...]
    bb = bb_ref[...]
    ae0 = jnp.dot(asrc[0], em, preferred_element_type=jnp.float32)
    ae1 = jnp.dot(asrc[1], em, preferred_element_type=jnp.float32)
    z0 = _lrelu(ae0[:, :, None] + bb[0][None, :, :])
    z1 = _lrelu(ae1[:, :, None] + bb[1][None, :, :])
    a0 = 1.0 / (1.0 + jnp.exp(z1 - z0))
    a_ref[0] = a0
    a_ref[1] = 1.0 - a0

    acc0 = jnp.zeros((_V, _FP), jnp.float32)
    acc1 = jnp.zeros((_V, _FP), jnp.float32)
    for h in range(_H):
        zh0 = _lrelu(asrc[0, :, h][:, None] + adt_ref[h, 0:_N][None, :])
        zh1 = _lrelu(asrc[1, :, h][:, None] + adt_ref[h, _N:2 * _N][None, :])
        ah0 = 1.0 / (1.0 + jnp.exp(zh1 - zh0))
        ah1 = 1.0 - ah0
        fd0 = fd_ref[0, :, _FP * h:_FP * (h + 1)]
        fd1 = fd_ref[1, :, _FP * h:_FP * (h + 1)]
        acc0 = acc0 + jax.nn.relu(
            jnp.dot(ah0, fd0, preferred_element_type=jnp.float32))
        acc1 = acc1 + jax.nn.relu(
            jnp.dot(ah1, fd1, preferred_element_type=jnp.float32))
    o_ref[0] = acc0[:, :_F]
    o_ref[1] = acc1[:, :_F]


def _main_kernel(asrc3, bb, adt, featd3, em):
    return pl.pallas_call(
        _main_body,
        grid=(_N // _V,),
        in_specs=[
            pl.BlockSpec((_C, _V, _H), lambda i: (0, i, 0)),
            pl.BlockSpec((_C, _R, 128), lambda i: (0, 0, 0)),
            pl.BlockSpec((_H, _CN), lambda i: (0, 0)),
            pl.BlockSpec((_C, _N, _H * _FP), lambda i: (0, 0, 0)),
            pl.BlockSpec((_H, _R), lambda i: (0, 0)),
        ],
        out_specs=[
            pl.BlockSpec((_C, _V, _R, 128), lambda i: (0, i, 0, 0)),
            pl.BlockSpec((_C, _V, _F), lambda i: (0, i, 0)),
        ],
        out_shape=[
            jax.ShapeDtypeStruct((_C, _N, _R, 128), jnp.float32),
            jax.ShapeDtypeStruct((_C, _N, _F), jnp.float32),
        ],
    )(asrc3, bb, adt, featd3, em)


def kernel(edge_idx, node_feats, fc, attn_src, attn_dst):
    edge_idx = edge_idx.astype(jnp.int32)

    partials = _sc_mask_partials(edge_idx)

    h_src = node_feats.reshape(_CN, _NT, _IF)
    feat4 = _feat_kernel(h_src, fc)
    flat = feat4.reshape(_CN, _NT * _IF)

    p2, s2 = _perm_matrices()
    asrc, adst, adt, featd3 = _prep_kernel(
        partials, flat, p2, s2, _pad_attn(attn_src), _pad_attn(attn_dst))

    asrc3 = asrc.reshape(_C, _N, _H)
    bb = adst.reshape(_C, _UT, 128, _H).transpose(0, 1, 3, 2).reshape(
        _C, _R, 128)
    featd3 = featd3.reshape(_C, _N, _H * _FP)

    em = (jax.lax.broadcasted_iota(jnp.int32, (_H, _R), 1) % _H
          == jax.lax.broadcasted_iota(jnp.int32, (_H, _R), 0)
          ).astype(jnp.float32)

    a4, outs = _main_kernel(asrc3, bb, adt, featd3, em)

    a = a4.reshape(_C, _N, _UT, _H, 128).transpose(0, 1, 2, 4, 3).reshape(
        _C, _N, _N, _H)
    outs4 = outs.reshape(_C, _N, _NT, _OF)
    return (outs4, a)


# speedup vs baseline: 41.2252x; 1.0520x over previous
import functools

import jax
import jax.numpy as jnp
import numpy as np
from jax import lax
from jax.experimental import pallas as pl
from jax.experimental.pallas import tpu as pltpu
from jax.experimental.pallas import tpu_sc as plsc

_C = 2
_N = 2048
_NT = 3
_IF = 128
_OF = 32
_H = 4
_CN = _C * _N
_F = _NT * _OF
_FP = 128
_K = _N * _H
_E = 65536

_NC = 2
_NS = 16
_NW = _NC * _NS
_EPW = _E // _NW

_V = 128


def _perm_matrices():
    p2 = np.zeros((_NT * _IF, _H * _FP), np.float32)
    for h in range(_H):
        for nt in range(_NT):
            for of in range(_OF):
                src = nt * 128 + h * 32 + of
                dst = h * _FP + nt * _OF + of
                p2[src, dst] = 1.0
    s2 = np.zeros((_H * _FP, _H), np.float32)
    for j in range(_H * _FP):
        s2[j, j // _FP] = 1.0
    return jnp.asarray(p2), jnp.asarray(s2)


def _pad_attn(attn):
    return jnp.pad(attn.reshape(1, _H, _F), ((0, 0), (0, 0), (0, _FP - _F))
                   ).reshape(1, _H * _FP)


def _sc_masks_body(edge_hbm, out_hbm, idx_s, idx_d, fs, fd):
    wid = lax.axis_index("s") * _NC + lax.axis_index("c")
    base = wid * _EPW
    pltpu.sync_copy(edge_hbm.at[0, pl.ds(base, _EPW)], idx_s)
    pltpu.sync_copy(edge_hbm.at[1, pl.ds(base, _EPW)], idx_d)

    zeros = jnp.zeros((16,), jnp.float32)

    def zbody(i, carry):
        fs[pl.ds(i * 16, 16)] = zeros
        fd[pl.ds(i * 16, 16)] = zeros
        return carry

    lax.fori_loop(0, _CN // 16, zbody, 0)

    ones = jnp.ones((16,), jnp.float32)

    def sbody(i, carry):
        vi = idx_s[pl.ds(i * 16, 16)]
        plsc.store_scatter(fs, [vi], ones)
        vj = idx_d[pl.ds(i * 16, 16)]
        plsc.store_scatter(fd, [vj], ones)
        return carry

    lax.fori_loop(0, _EPW // 16, sbody, 0)

    pltpu.sync_copy(fs, out_hbm.at[0, wid])
    pltpu.sync_copy(fd, out_hbm.at[1, wid])


def _sc_mask_partials(edge_idx):
    mesh = plsc.VectorSubcoreMesh(core_axis_name="c", subcore_axis_name="s")
    fn = functools.partial(
        pl.kernel,
        out_type=jax.ShapeDtypeStruct((2, _NW, _CN), jnp.float32),
        mesh=mesh,
        scratch_types=[
            pltpu.VMEM((_EPW,), jnp.int32),
            pltpu.VMEM((_EPW,), jnp.int32),
            pltpu.VMEM((_CN,), jnp.float32),
            pltpu.VMEM((_CN,), jnp.float32),
        ],
        compiler_params=pltpu.CompilerParams(needs_layout_passes=False),
    )(_sc_masks_body)
    return fn(edge_idx)


def _feat_body(h_ref, fc_ref, o_ref):
    o_ref[0, 0] = jnp.dot(
        h_ref[0, 0], fc_ref[0], preferred_element_type=jnp.float32
    )


def _feat_kernel(ht, fc):
    return pl.pallas_call(
        _feat_body,
        grid=(_C * _NT,),
        in_specs=[
            pl.BlockSpec((1, 1, _N, _IF), lambda i: (i // _NT, i % _NT, 0, 0)),
            pl.BlockSpec((1, _IF, _OF * _H), lambda i: (i % _NT, 0, 0)),
        ],
        out_specs=pl.BlockSpec((1, 1, _N, _OF * _H),
                               lambda i: (i // _NT, i % _NT, 0, 0)),
        out_shape=jax.ShapeDtypeStruct((_C, _NT, _N, _OF * _H), jnp.float32),
    )(ht, fc)


def _prep_body(part_ref, f_ref, p2_ref, s2_ref, as_ref, ad_ref,
               asrc_ref, adst_ref, adt_ref, fd_ref):
    part = part_ref[...]
    cnt = part.sum(axis=1)
    sflag = (cnt[0] > 0.0).astype(jnp.float32)
    dflag = (cnt[1] > 0.0).astype(jnp.float32)
    fp = jnp.dot(f_ref[...], p2_ref[...],
                 preferred_element_type=jnp.float32)
    ps = jnp.clip(fp * as_ref[...], -1e9, 1e9)
    asrc_ref[...] = jnp.dot(ps, s2_ref[...],
                            preferred_element_type=jnp.float32) * sflag[:, None]
    pd = jnp.clip(fp * ad_ref[...], -1e9, 1e9)
    adst = jnp.dot(pd, s2_ref[...],
                   preferred_element_type=jnp.float32) * dflag[:, None]
    adst_ref[...] = adst
    adt_ref[...] = adst.T
    fd_ref[...] = fp * dflag[:, None]


def _prep_kernel(partials, flat, p2, s2, attn_src_p, attn_dst_p):
    blk = 1024
    kp = _H * _FP
    return pl.pallas_call(
        _prep_body,
        grid=(_CN // blk,),
        in_specs=[
            pl.BlockSpec((2, _NW, blk), lambda i: (0, 0, i)),
            pl.BlockSpec((blk, _NT * _IF), lambda i: (i, 0)),
            pl.BlockSpec((_NT * _IF, kp), lambda i: (0, 0)),
            pl.BlockSpec((kp, _H), lambda i: (0, 0)),
            pl.BlockSpec((1, kp), lambda i: (0, 0)),
            pl.BlockSpec((1, kp), lambda i: (0, 0)),
        ],
        out_specs=[
            pl.BlockSpec((blk, _H), lambda i: (i, 0)),
            pl.BlockSpec((blk, _H), lambda i: (i, 0)),
            pl.BlockSpec((_H, blk), lambda i: (0, i)),
            pl.BlockSpec((blk, kp), lambda i: (i, 0)),
        ],
        out_shape=[
            jax.ShapeDtypeStruct((_CN, _H), jnp.float32),
            jax.ShapeDtypeStruct((_CN, _H), jnp.float32),
            jax.ShapeDtypeStruct((_H, _CN), jnp.float32),
            jax.ShapeDtypeStruct((_CN, kp), jnp.float32),
        ],
    )(partials, flat, p2, s2, attn_src_p, attn_dst_p)


def _lrelu(z):
    return jnp.maximum(z, 0.2 * z)


_UT = _N // 128
_R = _UT * _H


def _main_body(asrc_ref, bb_ref, adt_ref, fd_ref, e_ref, a_ref, o_ref):
    asrc = asrc_ref[...]
    em = e_ref[...]
    bb = bb_ref[...]
    ae0 = jnp.dot(asrc[0], em, preferred_element_type=jnp.float32)
    ae1 = jnp.dot(asrc[1], em, preferred_element_type=jnp.float32)
    z0 = _lrelu(ae0[:, :, None] + bb[0][None, :, :])
    z1 = _lrelu(ae1[:, :, None] + bb[1][None, :, :])
    a0 = 1.0 / (1.0 + jnp.exp(z1 - z0))
    a_ref[0] = a0
    a_ref[1] = 1.0 - a0

    acc0 = jnp.zeros((_V, _FP), jnp.float32)
    acc1 = jnp.zeros((_V, _FP), jnp.float32)
    for h in range(_H):
        zh0 = _lrelu(asrc[0, :, h][:, None] + adt_ref[h, 0:_N][None, :])
        zh1 = _lrelu(asrc[1, :, h][:, None] + adt_ref[h, _N:2 * _N][None, :])
        ah0 = 1.0 / (1.0 + jnp.exp(zh1 - zh0))
        ah1 = 1.0 - ah0
        fd0 = fd_ref[0, :, _FP * h:_FP * (h + 1)]
        fd1 = fd_ref[1, :, _FP * h:_FP * (h + 1)]
        acc0 = acc0 + jax.nn.relu(
            jnp.dot(ah0, fd0, preferred_element_type=jnp.float32))
        acc1 = acc1 + jax.nn.relu(
            jnp.dot(ah1, fd1, preferred_element_type=jnp.float32))
    o_ref[0] = acc0[:, :_F]
    o_ref[1] = acc1[:, :_F]


def _main_kernel(asrc3, bb, adt, featd3, em):
    return pl.pallas_call(
        _main_body,
        grid=(_N // _V,),
        in_specs=[
            pl.BlockSpec((_C, _V, _H), lambda i: (0, i, 0)),
            pl.BlockSpec((_C, _R, 128), lambda i: (0, 0, 0)),
            pl.BlockSpec((_H, _CN), lambda i: (0, 0)),
            pl.BlockSpec((_C, _N, _H * _FP), lambda i: (0, 0, 0)),
            pl.BlockSpec((_H, _R), lambda i: (0, 0)),
        ],
        out_specs=[
            pl.BlockSpec((_C, _V, _R, 128), lambda i: (0, i, 0, 0)),
            pl.BlockSpec((_C, _V, _F), lambda i: (0, i, 0)),
        ],
        out_shape=[
            jax.ShapeDtypeStruct((_C, _N, _R, 128), jnp.float32),
            jax.ShapeDtypeStruct((_C, _N, _F), jnp.float32),
        ],
    )(asrc3, bb, adt, featd3, em)


def kernel(edge_idx, node_feats, fc, attn_src, attn_dst):
    edge_idx = edge_idx.astype(jnp.int32)

    partials = _sc_mask_partials(edge_idx)

    ht = node_feats.transpose(0, 2, 1, 3)
    feat4 = _feat_kernel(ht, fc)
    flat = feat4.reshape(_CN, _NT * _IF)

    p2, s2 = _perm_matrices()
    asrc, adst, adt, featd3 = _prep_kernel(
        partials, flat, p2, s2, _pad_attn(attn_src), _pad_attn(attn_dst))

    asrc3 = asrc.reshape(_C, _N, _H)
    bb = adst.reshape(_C, _UT, 128, _H).transpose(0, 1, 3, 2).reshape(
        _C, _R, 128)
    featd3 = featd3.reshape(_C, _N, _H * _FP)

    em = (jax.lax.broadcasted_iota(jnp.int32, (_H, _R), 1) % _H
          == jax.lax.broadcasted_iota(jnp.int32, (_H, _R), 0)
          ).astype(jnp.float32)

    a4, outs = _main_kernel(asrc3, bb, adt, featd3, em)

    a = a4.reshape(_C, _N, _UT, _H, 128).transpose(0, 1, 2, 4, 3).reshape(
        _C, _N, _N, _H)
    outs4 = outs.reshape(_C, _N, _NT, _OF)
    return (outs4, a)


# speedup vs baseline: 42.7092x; 1.0360x over previous
import functools

import jax
import jax.numpy as jnp
import numpy as np
from jax import lax
from jax.experimental import pallas as pl
from jax.experimental.pallas import tpu as pltpu
from jax.experimental.pallas import tpu_sc as plsc

_C = 2
_N = 2048
_NT = 3
_IF = 128
_OF = 32
_H = 4
_CN = _C * _N
_F = _NT * _OF
_FP = 128
_K = _N * _H
_E = 65536

_NC = 2
_NS = 16
_NW = _NC * _NS
_EPW = _E // _NW

_V = 128


def _perm_matrices():
    p2 = np.zeros((_NT * _IF, _H * _FP), np.float32)
    for h in range(_H):
        for nt in range(_NT):
            for of in range(_OF):
                src = nt * 128 + h * 32 + of
                dst = h * _FP + nt * _OF + of
                p2[src, dst] = 1.0
    s2 = np.zeros((_H * _FP, _H), np.float32)
    for j in range(_H * _FP):
        s2[j, j // _FP] = 1.0
    return jnp.asarray(p2.reshape(_NT, _IF, _H * _FP)), jnp.asarray(s2)


def _pad_attn(attn):
    return jnp.pad(attn.reshape(1, _H, _F), ((0, 0), (0, 0), (0, _FP - _F))
                   ).reshape(1, _H * _FP)


def _sc_masks_body(edge_hbm, out_hbm, idx_s, idx_d, fs, fd):
    wid = lax.axis_index("s") * _NC + lax.axis_index("c")
    base = wid * _EPW
    pltpu.sync_copy(edge_hbm.at[0, pl.ds(base, _EPW)], idx_s)
    pltpu.sync_copy(edge_hbm.at[1, pl.ds(base, _EPW)], idx_d)

    zeros = jnp.zeros((16,), jnp.float32)

    def zbody(i, carry):
        fs[pl.ds(i * 16, 16)] = zeros
        fd[pl.ds(i * 16, 16)] = zeros
        return carry

    lax.fori_loop(0, _CN // 16, zbody, 0)

    ones = jnp.ones((16,), jnp.float32)

    def sbody(i, carry):
        vi = idx_s[pl.ds(i * 16, 16)]
        plsc.store_scatter(fs, [vi], ones)
        vj = idx_d[pl.ds(i * 16, 16)]
        plsc.store_scatter(fd, [vj], ones)
        return carry

    lax.fori_loop(0, _EPW // 16, sbody, 0)

    pltpu.sync_copy(fs, out_hbm.at[0, wid])
    pltpu.sync_copy(fd, out_hbm.at[1, wid])


def _sc_mask_partials(edge_idx):
    mesh = plsc.VectorSubcoreMesh(core_axis_name="c", subcore_axis_name="s")
    fn = functools.partial(
        pl.kernel,
        out_type=jax.ShapeDtypeStruct((2, _NW, _CN), jnp.float32),
        mesh=mesh,
        scratch_types=[
            pltpu.VMEM((_EPW,), jnp.int32),
            pltpu.VMEM((_EPW,), jnp.int32),
            pltpu.VMEM((_CN,), jnp.float32),
            pltpu.VMEM((_CN,), jnp.float32),
        ],
        compiler_params=pltpu.CompilerParams(needs_layout_passes=False),
    )(_sc_masks_body)
    return fn(edge_idx)


def _feat_body(h_ref, fc_ref, o_ref):
    o_ref[0, 0] = jnp.dot(
        h_ref[0, 0], fc_ref[0], preferred_element_type=jnp.float32
    )


def _feat_kernel(ht, fc):
    return pl.pallas_call(
        _feat_body,
        grid=(_C * _NT,),
        in_specs=[
            pl.BlockSpec((1, 1, _N, _IF), lambda i: (i // _NT, i % _NT, 0, 0)),
            pl.BlockSpec((1, _IF, _OF * _H), lambda i: (i % _NT, 0, 0)),
        ],
        out_specs=pl.BlockSpec((1, 1, _N, _OF * _H),
                               lambda i: (i // _NT, i % _NT, 0, 0)),
        out_shape=jax.ShapeDtypeStruct((_C, _NT, _N, _OF * _H), jnp.float32),
    )(ht, fc)


def _prep_body(part_ref, f_ref, p2_ref, s2_ref, as_ref, ad_ref,
               asrc_ref, adst_ref, adt_ref, fd_ref):
    part = part_ref[...]
    cnt = part.sum(axis=1)
    sflag = (cnt[0] > 0.0).astype(jnp.float32)
    dflag = (cnt[1] > 0.0).astype(jnp.float32)
    x = f_ref[...].reshape(-1, _NT, _IF)
    fp = (jnp.dot(x[:, 0], p2_ref[0], preferred_element_type=jnp.float32)
          + jnp.dot(x[:, 1], p2_ref[1], preferred_element_type=jnp.float32)
          + jnp.dot(x[:, 2], p2_ref[2], preferred_element_type=jnp.float32))
    ps = jnp.clip(fp * as_ref[...], -1e9, 1e9)
    asrc_ref[...] = jnp.dot(ps, s2_ref[...],
                            preferred_element_type=jnp.float32) * sflag[:, None]
    pd = jnp.clip(fp * ad_ref[...], -1e9, 1e9)
    adst = jnp.dot(pd, s2_ref[...],
                   preferred_element_type=jnp.float32) * dflag[:, None]
    adst_ref[...] = adst
    adt_ref[...] = adst.T
    fd_ref[...] = fp * dflag[:, None]


def _prep_kernel(partials, flat, p2, s2, attn_src_p, attn_dst_p):
    blk = 1024
    kp = _H * _FP
    return pl.pallas_call(
        _prep_body,
        grid=(_CN // blk,),
        in_specs=[
            pl.BlockSpec((2, _NW, blk), lambda i: (0, 0, i)),
            pl.BlockSpec((_NT * blk, _IF), lambda i: (i, 0)),
            pl.BlockSpec((_NT, _IF, kp), lambda i: (0, 0, 0)),
            pl.BlockSpec((kp, _H), lambda i: (0, 0)),
            pl.BlockSpec((1, kp), lambda i: (0, 0)),
            pl.BlockSpec((1, kp), lambda i: (0, 0)),
        ],
        out_specs=[
            pl.BlockSpec((blk, _H), lambda i: (i, 0)),
            pl.BlockSpec((blk, _H), lambda i: (i, 0)),
            pl.BlockSpec((_H, blk), lambda i: (0, i)),
            pl.BlockSpec((blk, kp), lambda i: (i, 0)),
        ],
        out_shape=[
            jax.ShapeDtypeStruct((_CN, _H), jnp.float32),
            jax.ShapeDtypeStruct((_CN, _H), jnp.float32),
            jax.ShapeDtypeStruct((_H, _CN), jnp.float32),
            jax.ShapeDtypeStruct((_CN, kp), jnp.float32),
        ],
    )(partials, flat, p2, s2, attn_src_p, attn_dst_p)


def _lrelu(z):
    return jnp.maximum(z, 0.2 * z)


_UT = _N // 128
_R = _UT * _H


def _main_body(asrc_ref, bb_ref, adt_ref, fd_ref, e_ref, a_ref, o_ref):
    asrc = asrc_ref[...]
    em = e_ref[...]
    bb = bb_ref[...]
    ae0 = jnp.dot(asrc[0], em, preferred_element_type=jnp.float32)
    ae1 = jnp.dot(asrc[1], em, preferred_element_type=jnp.float32)
    z0 = _lrelu(ae0[:, :, None] + bb[0][None, :, :])
    z1 = _lrelu(ae1[:, :, None] + bb[1][None, :, :])
    a0 = 1.0 / (1.0 + jnp.exp(z1 - z0))
    a_ref[0] = a0
    a_ref[1] = 1.0 - a0

    acc0 = jnp.zeros((_V, _FP), jnp.float32)
    acc1 = jnp.zeros((_V, _FP), jnp.float32)
    for h in range(_H):
        zh0 = _lrelu(asrc[0, :, h][:, None] + adt_ref[h, 0:_N][None, :])
        zh1 = _lrelu(asrc[1, :, h][:, None] + adt_ref[h, _N:2 * _N][None, :])
        ah0 = 1.0 / (1.0 + jnp.exp(zh1 - zh0))
        ah1 = 1.0 - ah0
        fd0 = fd_ref[0, :, _FP * h:_FP * (h + 1)]
        fd1 = fd_ref[1, :, _FP * h:_FP * (h + 1)]
        acc0 = acc0 + jax.nn.relu(
            jnp.dot(ah0, fd0, preferred_element_type=jnp.float32))
        acc1 = acc1 + jax.nn.relu(
            jnp.dot(ah1, fd1, preferred_element_type=jnp.float32))
    o_ref[0] = acc0[:, :_F]
    o_ref[1] = acc1[:, :_F]


def _main_kernel(asrc3, bb, adt, featd3, em):
    return pl.pallas_call(
        _main_body,
        grid=(_N // _V,),
        in_specs=[
            pl.BlockSpec((_C, _V, _H), lambda i: (0, i, 0)),
            pl.BlockSpec((_C, _R, 128), lambda i: (0, 0, 0)),
            pl.BlockSpec((_H, _CN), lambda i: (0, 0)),
            pl.BlockSpec((_C, _N, _H * _FP), lambda i: (0, 0, 0)),
            pl.BlockSpec((_H, _R), lambda i: (0, 0)),
        ],
        out_specs=[
            pl.BlockSpec((_C, _V, _R, 128), lambda i: (0, i, 0, 0)),
            pl.BlockSpec((_C, _V, _F), lambda i: (0, i, 0)),
        ],
        out_shape=[
            jax.ShapeDtypeStruct((_C, _N, _R, 128), jnp.float32),
            jax.ShapeDtypeStruct((_C, _N, _F), jnp.float32),
        ],
    )(asrc3, bb, adt, featd3, em)


def kernel(edge_idx, node_feats, fc, attn_src, attn_dst):
    edge_idx = edge_idx.astype(jnp.int32)

    partials = _sc_mask_partials(edge_idx)

    ht = node_feats.transpose(0, 2, 1, 3)
    feat4 = _feat_kernel(ht, fc)
    flat = feat4.reshape(_CN * _NT, _IF)

    p2, s2 = _perm_matrices()
    asrc, adst, adt, featd3 = _prep_kernel(
        partials, flat, p2, s2, _pad_attn(attn_src), _pad_attn(attn_dst))

    asrc3 = asrc.reshape(_C, _N, _H)
    bb = adst.reshape(_C, _UT, 128, _H).transpose(0, 1, 3, 2).reshape(
        _C, _R, 128)
    featd3 = featd3.reshape(_C, _N, _H * _FP)

    em = (jax.lax.broadcasted_iota(jnp.int32, (_H, _R), 1) % _H
          == jax.lax.broadcasted_iota(jnp.int32, (_H, _R), 0)
          ).astype(jnp.float32)

    a4, outs = _main_kernel(asrc3, bb, adt, featd3, em)

    a = a4.reshape(_C, _N, _UT, _H, 128).transpose(0, 1, 2, 4, 3).reshape(
        _C, _N, _N, _H)
    outs4 = outs.reshape(_C, _N, _NT, _OF)
    return (outs4, a)
